# baseline (speedup 1.0000x reference)
import jax
import jax.numpy as jnp
from jax import lax
from jax.experimental import pallas as pl
from jax.experimental.pallas import tpu as pltpu

NT = 64
NB = 16
TOT = 32768
CHUNK = 512
GROUP = 8
LN2 = 0.6931471805599453


def _crf_body(cu_ref, lens_ref, em_hbm, trans_ref, head_ref, tail_ref,
              out_ref, embuf, sem):
    E = jnp.exp(trans_ref[...]).astype(jnp.bfloat16)
    eh = jnp.exp(head_ref[...])
    et = jnp.exp(tail_ref[...])
    lens = lens_ref[...]

    def mx(b, m):
        return jnp.maximum(m, cu_ref[b + 1] - cu_ref[b])
    maxlen = lax.fori_loop(0, NB, mx, jnp.int32(0))
    nch = lax.div(maxlen + (CHUNK - 1), CHUNK)

    def issue(ci, buf):
        for b in range(NB):
            off = jnp.minimum(cu_ref[b] + ci * CHUNK, TOT - CHUNK)
            pltpu.make_async_copy(
                em_hbm.at[pl.ds(off, CHUNK), :],
                embuf.at[buf, :, b, :],
                sem.at[buf],
            ).start()

    def wait(buf):
        for b in range(NB):
            pltpu.make_async_copy(
                em_hbm.at[pl.ds(0, CHUNK), :],
                embuf.at[buf, :, b, :],
                sem.at[buf],
            ).wait()

    issue(0, 0)

    def chunk_body(ci, carry):
        buf = lax.rem(ci, 2)

        @pl.when(ci + 1 < nch)
        def _():
            issue(ci + 1, 1 - buf)

        wait(buf)

        def group(g, c2):
            qs = [c2[0], c2[1]]
            cfs = [c2[2], c2[3]]
            psn = [c2[4], c2[5]]
            cfsn = [c2[6], c2[7]]
            lh = [lens[0:8], lens[8:16]]
            for k in range(GROUP):
                t = GROUP * g + k
                tg = ci * CHUNK + t
                eem = jnp.exp(embuf[buf, t])
                eh_ = [eem[0:8] * eh, eem[8:16] * eh]
                for h in range(2):
                    q0 = lax.dot_general(qs[h].astype(jnp.bfloat16), E,
                                         (((1,), (0,)), ((), ())),
                                         preferred_element_type=jnp.float32)
                    eslice = eem[0:8] if h == 0 else eem[8:16]
                    qs[h] = jnp.where(tg == 0, eh_[h], q0 * eslice)
                    hit = tg == (lh[h] - 1)
                    psn[h] = jnp.where(hit, qs[h], psn[h])
                    cfsn[h] = jnp.where(hit, cfs[h], cfsn[h])
            for h in range(2):
                m = jnp.max(qs[h], axis=1, keepdims=True)
                bits = lax.bitcast_convert_type(m, jnp.int32)
                ef = lax.shift_right_logical(bits, 23) & 0xFF
                scale = lax.bitcast_convert_type(
                    lax.shift_left(254 - ef, 23), jnp.float32)
                qs[h] = qs[h] * scale
                cfs[h] = cfs[h] + (ef - 127).astype(jnp.float32)
            return (qs[0], qs[1], cfs[0], cfs[1],
                    psn[0], psn[1], cfsn[0], cfsn[1])

        return lax.fori_loop(0, CHUNK // GROUP, group, carry, unroll=2)

    zv = jnp.zeros((NB // 2, NT), jnp.float32)
    zc = jnp.zeros((NB // 2, 1), jnp.float32)
    fin = lax.fori_loop(0, nch, chunk_body,
                        (zv, zv, zc, zc, zv, zv, zc, zc))
    psnap = jnp.concatenate([fin[4], fin[5]], axis=0)
    cfsnap = jnp.concatenate([fin[6], fin[7]], axis=0)
    s = jnp.sum(psnap * et, axis=1, keepdims=True)
    out_ref[...] = jnp.log(s) + cfsnap * LN2


def kernel(emissions, transitions, head_transitions, tail_transitions,
           cu_seqlens):
    em = emissions.reshape(TOT, NT)
    trans = transitions.reshape(NT, NT)
    head = head_transitions.reshape(1, NT)
    tail = tail_transitions.reshape(1, NT)
    cu = cu_seqlens.astype(jnp.int32)
    lens = (cu[1:] - cu[:-1]).reshape(NB, 1)
    return pl.pallas_call(
        _crf_body,
        out_shape=jax.ShapeDtypeStruct((NB, 1), jnp.float32),
        in_specs=[
            pl.BlockSpec(memory_space=pltpu.SMEM),
            pl.BlockSpec(memory_space=pltpu.VMEM),
            pl.BlockSpec(memory_space=pltpu.MemorySpace.HBM),
            pl.BlockSpec(memory_space=pltpu.VMEM),
            pl.BlockSpec(memory_space=pltpu.VMEM),
            pl.BlockSpec(memory_space=pltpu.VMEM),
        ],
        out_specs=pl.BlockSpec(memory_space=pltpu.VMEM),
        scratch_shapes=[
            pltpu.VMEM((2, CHUNK, NB, NT), jnp.float32),
            pltpu.SemaphoreType.DMA((2,)),
        ],
    )(cu, lens, em, trans, head, tail)
